# baseline (speedup 1.0000x reference)
import functools

import jax
import jax.numpy as jnp
from jax import lax
from jax.experimental import pallas as pl
from jax.experimental.pallas import tpu as pltpu
from jax.experimental.pallas import tpu_sc as plsc

NC = 2
NS = 16
NW = NC * NS
LANES = 16

CHUNK = 256


@functools.lru_cache(maxsize=None)
def _build(N: int, V: int, D: int):
    n_per_w = N // NW
    n_chunks = n_per_w // CHUNK
    n_groups = CHUNK // LANES
    mesh = plsc.VectorSubcoreMesh(core_axis_name="c", subcore_axis_name="s")

    @functools.partial(
        pl.kernel,
        mesh=mesh,
        compiler_params=pltpu.CompilerParams(needs_layout_passes=False),
        out_type=jax.ShapeDtypeStruct((N // 4, 128), jnp.float32),
        scratch_types=[
            pltpu.VMEM((n_per_w,), jnp.int32),
            pltpu.VMEM((CHUNK,), jnp.int32),
            pltpu.VMEM((CHUNK,), jnp.int32),
            pltpu.VMEM((CHUNK,), jnp.int32),
            pltpu.VMEM((CHUNK,), jnp.int32),
            pltpu.VMEM((CHUNK, 128), jnp.float32),
            pltpu.VMEM((CHUNK, 128), jnp.float32),
            pltpu.VMEM((CHUNK // 4, 128), jnp.float32),
            pltpu.VMEM((CHUNK // 4, 128), jnp.float32),
            pltpu.SemaphoreType.DMA,
            pltpu.SemaphoreType.DMA,
            pltpu.SemaphoreType.DMA,
            pltpu.SemaphoreType.DMA,
        ],
    )
    def gather_kernel(table_hbm, idx_hbm, out_hbm,
                      idx_v, blk0, blk1, sub0, sub1,
                      pad0, pad1, cmp0, cmp1,
                      g0, g1, w0, w1):
        wid = lax.axis_index("s") * NC + lax.axis_index("c")
        w_base = pl.multiple_of(wid * n_per_w, n_per_w)
        blk = (blk0, blk1)
        sub = (sub0, sub1)
        pad = (pad0, pad1)
        cmp = (cmp0, cmp1)
        gsem = (g0, g1)
        wsem = (w0, w1)

        pltpu.sync_copy(idx_hbm.at[pl.ds(w_base, n_per_w)], idx_v)

        def compute_ids(i, b):
            for k in range(n_groups):
                v = idx_v[pl.ds(i * CHUNK + k * LANES, LANES)]
                blk[b][pl.ds(k * LANES, LANES)] = lax.shift_right_logical(v, 2)
                sub[b][pl.ds(k * LANES, LANES)] = lax.bitwise_and(v, 3)

        def start_gather(b):
            return pltpu.async_copy(table_hbm.at[blk[b]], pad[b], gsem[b])

        def extract(b):
            lane = lax.iota(jnp.int32, LANES)

            def grp(k, carry):
                s = sub[b][pl.ds(k * LANES, LANES)]
                rows = lane + k * LANES
                scol = s * 32
                drow = lax.shift_right_logical(rows, 2)
                dcol0 = lax.bitwise_and(rows, 3) * 32
                for c in range(32):
                    vals = plsc.load_gather(pad[b], [rows, scol + c])
                    plsc.store_scatter(cmp[b], [drow, dcol0 + c], vals)
                return carry

            lax.fori_loop(0, n_groups, grp, 0, unroll=False)

        def chunk_step(i, b, start_next):
            pltpu.make_async_copy(
                table_hbm.at[pl.ds(0, CHUNK)], pad[b], gsem[b]).wait()

            @pl.when(i >= 2)
            def _():
                pltpu.make_async_copy(
                    cmp[b], out_hbm.at[pl.ds(0, CHUNK // 4)], wsem[b]).wait()

            extract(b)

            @pl.when(start_next)
            def _():
                compute_ids(i + 2, b)
                start_gather(b)

            out_rows = CHUNK // 4
            out_off = pl.multiple_of((w_base + i * CHUNK) // 4, out_rows)
            pltpu.async_copy(
                cmp[b], out_hbm.at[pl.ds(out_off, out_rows)], wsem[b])

        compute_ids(0, 0)
        start_gather(0)
        compute_ids(1, 1)
        start_gather(1)

        def body(j, carry):
            i = j * 2
            chunk_step(i, 0, i + 2 < n_chunks)
            chunk_step(i + 1, 1, i + 3 < n_chunks)
            return carry

        lax.fori_loop(0, n_chunks // 2, body, 0, unroll=False)
        pltpu.make_async_copy(
            cmp[0], out_hbm.at[pl.ds(0, CHUNK // 4)], wsem[0]).wait()
        pltpu.make_async_copy(
            cmp[1], out_hbm.at[pl.ds(0, CHUNK // 4)], wsem[1]).wait()

    return gather_kernel


def kernel(x, table):
    Bq, Lq = x.shape
    V, D = table.shape
    N = Bq * Lq
    idx = x.reshape(N).astype(jnp.int32)
    table4 = table.reshape(V // 4, 4 * D)
    out4 = _build(N, V, D)(table4, idx)
    return out4.reshape(N, D).reshape(Bq, Lq, D)


# speedup vs baseline: 1.9526x; 1.9526x over previous
import functools

import jax
import jax.numpy as jnp
from jax import lax
from jax.experimental import pallas as pl
from jax.experimental.pallas import tpu as pltpu
from jax.experimental.pallas import tpu_sc as plsc

NC = 2
NS = 16
NW = NC * NS

CHUNK = 1600
SPLIT = 4


@functools.lru_cache(maxsize=None)
def _build(N: int, V: int, D: int):
    n_per_w = N // NW
    n_chunks = n_per_w // CHUNK
    mesh = plsc.VectorSubcoreMesh(core_axis_name="c", subcore_axis_name="s")

    @functools.partial(
        pl.kernel,
        mesh=mesh,
        compiler_params=pltpu.CompilerParams(use_tc_tiling_on_sc=False),
        out_type=jax.ShapeDtypeStruct((N, D), jnp.float32),
        scratch_types=[
            pltpu.VMEM((n_per_w,), jnp.int32),
            pltpu.VMEM((CHUNK, D), jnp.float32),
            pltpu.VMEM((CHUNK, D), jnp.float32),
            pltpu.SemaphoreType.DMA,
            pltpu.SemaphoreType.DMA,
            pltpu.SemaphoreType.DMA,
            pltpu.SemaphoreType.DMA,
        ],
    )
    def gather_kernel(table_hbm, idx_hbm, out_hbm,
                      idx_v, rows0, rows1, g0, g1, o0, o1):
        wid = lax.axis_index("s") * NC + lax.axis_index("c")
        w_base = wid * n_per_w
        rows = (rows0, rows1)
        gsem = (g0, g1)
        osem = (o0, o1)

        pltpu.sync_copy(idx_hbm.at[pl.ds(w_base, n_per_w)], idx_v)

        def gather(i, b):
            return pltpu.async_copy(
                table_hbm.at[idx_v.at[pl.ds(i * CHUNK, CHUNK)]],
                rows[b], gsem[b])

        def writeback(i, b):
            return pltpu.async_copy(
                rows[b], out_hbm.at[pl.ds(w_base + i * CHUNK, CHUNK)],
                osem[b])

        pending_g = gather(0, 0)
        pending_o = [None, None]
        for i in range(n_chunks):
            b = i % 2
            pending_g.wait()
            if i + 1 < n_chunks:
                if pending_o[1 - b] is not None:
                    pending_o[1 - b].wait()
                pending_g = gather(i + 1, 1 - b)
            pending_o[b] = writeback(i, b)
        for p in pending_o:
            if p is not None:
                p.wait()

    return gather_kernel


def kernel(x, table):
    Bq, Lq = x.shape
    V, D = table.shape
    N = Bq * Lq
    idx = x.reshape(N).astype(jnp.int32)
    npiece = N // SPLIT
    bpiece = Bq // SPLIT
    gk = _build(npiece, V, D)
    pieces = [
        gk(table, lax.slice(idx, (i * npiece,), ((i + 1) * npiece,)))
        .reshape(bpiece, Lq, D)
        for i in range(SPLIT)
    ]
    return jnp.concatenate(pieces, axis=0)
